# baseline (speedup 1.0000x reference)
import functools

import jax
import jax.numpy as jnp
from jax import lax
from jax.experimental import pallas as pl
from jax.experimental.pallas import tpu as pltpu
from jax.experimental.pallas import tpu_sc as plsc

DIM = 64
BATCH = 16384
NROW = 1000000
NC = 2
NS = 16
NW = NC * NS
BPW = BATCH // NW
L = 16
NG = BPW // L
RPL = NROW // L


def _body(uid_hbm, iid_hbm, utab_hbm, itab_hbm, out_hbm,
          uids_v, iids_v, idx_v, rows_v, out_v, sem):
    wid = lax.axis_index("s") * NC + lax.axis_index("c")
    base = wid * BPW

    pltpu.sync_copy(uid_hbm.at[pl.ds(base, BPW)], uids_v)
    pltpu.sync_copy(iid_hbm.at[pl.ds(base, BPW)], iids_v)

    lane = lax.iota(jnp.int32, L)
    drow = [(jnp.full((L,), 16 * q, jnp.int32) + lane) * RPL
            for q in range(DIM // L)]

    def xlane(v, t):
        return v.at[lane ^ t].get(mode="promise_in_bounds")

    def splat(s):
        return jnp.full((L,), 0, jnp.int32) + s

    def group(g, _):
        uvec = uids_v[pl.ds(g * L, L)]
        ivec = iids_v[pl.ds(g * L, L)]
        for k in range(L):
            for t, vec, tab in ((0, uvec, utab_hbm), (1, ivec, itab_hbm)):
                rbase = splat(vec[k] >> 4)
                for q in range(DIM // L):
                    idx_v[k, t, pl.ds(q * L, L)] = rbase + drow[q]
                pltpu.async_copy(tab.at[idx_v.at[k, t]], rows_v.at[k, t], sem)
        for k in range(L):
            pltpu.make_async_copy(utab_hbm.at[idx_v.at[k, 0]],
                                  rows_v.at[k, 0], sem).wait()
            pltpu.make_async_copy(itab_hbm.at[idx_v.at[k, 1]],
                                  rows_v.at[k, 1], sem).wait()
        vecs = []
        for k in range(L):
            ulane = splat(uvec[k] & 15)
            ilane = splat(ivec[k] & 15)
            acc = None
            for q in range(DIM // L):
                rq = jnp.full((L,), q * L, jnp.int32) + lane
                gu = plsc.load_gather(rows_v, [splat(k), splat(0), rq, ulane])
                gi = plsc.load_gather(rows_v, [splat(k), splat(1), rq, ilane])
                prod = gu * gi
                acc = prod if acc is None else acc + prod
            vecs.append(acc)
        t = 1
        while len(vecs) > 1:
            m = (lane & t) != 0
            vecs = [jnp.where(m, vecs[i + 1] + xlane(vecs[i + 1], t),
                              vecs[i] + xlane(vecs[i], t))
                    for i in range(0, len(vecs), 2)]
            t *= 2
        out_v[pl.ds(g * L, L)] = vecs[0]
        return _

    lax.fori_loop(0, NG, group, None)

    pltpu.sync_copy(out_v, out_hbm.at[pl.ds(base, BPW)])


@jax.jit
def _scores(user_ids, item_ids, utab_lin, itab_lin):
    mesh = plsc.VectorSubcoreMesh(core_axis_name="c", subcore_axis_name="s")
    kern = functools.partial(
        pl.kernel,
        out_type=jax.ShapeDtypeStruct((BATCH,), jnp.float32),
        mesh=mesh,
        compiler_params=pltpu.CompilerParams(use_tc_tiling_on_sc=False,
                                             needs_layout_passes=False),
        scratch_types=[
            pltpu.VMEM((BPW,), jnp.int32),
            pltpu.VMEM((BPW,), jnp.int32),
            pltpu.VMEM((L, 2, DIM), jnp.int32),
            pltpu.VMEM((L, 2, DIM, L), jnp.float32),
            pltpu.VMEM((BPW,), jnp.float32),
            pltpu.SemaphoreType.DMA,
        ],
    )(_body)
    return kern(user_ids, item_ids, utab_lin, itab_lin)


def kernel(user_ids, item_ids, user_table, item_table):
    return _scores(user_ids.astype(jnp.int32), item_ids.astype(jnp.int32),
                   user_table.T.reshape(DIM * NROW // L, L),
                   item_table.T.reshape(DIM * NROW // L, L))


# speedup vs baseline: 25.8880x; 25.8880x over previous
import functools

import jax
import jax.numpy as jnp
from jax import lax
from jax.experimental import pallas as pl
from jax.experimental.pallas import tpu as pltpu
from jax.experimental.pallas import tpu_sc as plsc

DIM = 64
BATCH = 16384
NROW = 1000000
NFULL = (NROW // 128) * 128
BMAX = NROW // 128 - 1
NC = 2
NS = 16
NW = NC * NS
BPW = BATCH // NW
L = 16
NG = BPW // L
RING = 4


def _body(uid_hbm, iid_hbm, utab_hbm, itab_hbm, ulast_hbm, ilast_hbm, out_hbm,
          uids_v, iids_v, ublk_v, iblk_v, ulast_v, ilast_v, out_v, sem):
    wid = lax.axis_index("s") * NC + lax.axis_index("c")
    base = wid * BPW

    pltpu.sync_copy(uid_hbm.at[pl.ds(base, BPW)], uids_v)
    pltpu.sync_copy(iid_hbm.at[pl.ds(base, BPW)], iids_v)
    pltpu.sync_copy(ulast_hbm, ulast_v)
    pltpu.sync_copy(ilast_hbm, ilast_v)

    lane = lax.iota(jnp.int32, L)
    dvec = [jnp.full((L,), q * L, jnp.int32) + lane for q in range(DIM // L)]

    def xlane(v, t):
        return v.at[lane ^ t].get(mode="promise_in_bounds")

    def splat(s):
        return jnp.full((L,), 0, jnp.int32) + s

    def fire(vec, k, tab, blk):
        b = jnp.minimum(vec[k] >> 7, BMAX)
        off = pl.multiple_of(b * 128, 128)
        pltpu.async_copy(tab.at[:, pl.ds(off, 128)], blk.at[k % RING], sem)

    def drain(tab, blk, k):
        pltpu.make_async_copy(tab.at[:, pl.ds(0, 128)],
                              blk.at[k % RING], sem).wait()

    def col(vec, k, blk, lastf):
        vid = vec[k]
        c = splat(vid & 127)
        rel = splat(jnp.maximum(vid - NFULL, 0) * DIM)
        m = splat(vid) >= NFULL
        out = []
        for q in range(DIM // L):
            hbm_q = plsc.load_gather(blk, [splat(k % RING), dvec[q], c])
            last_q = plsc.load_gather(lastf, [rel + dvec[q]])
            out.append(jnp.where(m, last_q, hbm_q))
        return out

    def group(g, _):
        uvec = uids_v[pl.ds(g * L, L)]
        ivec = iids_v[pl.ds(g * L, L)]
        for k in range(RING):
            fire(uvec, k, utab_hbm, ublk_v)
            fire(ivec, k, itab_hbm, iblk_v)
        vecs = []
        for k in range(L):
            drain(utab_hbm, ublk_v, k)
            drain(itab_hbm, iblk_v, k)
            u = col(uvec, k, ublk_v, ulast_v)
            v = col(ivec, k, iblk_v, ilast_v)
            acc = None
            for q in range(DIM // L):
                prod = u[q] * v[q]
                acc = prod if acc is None else acc + prod
            vecs.append(acc)
            if k + RING < L:
                fire(uvec, k + RING, utab_hbm, ublk_v)
                fire(ivec, k + RING, itab_hbm, iblk_v)
        t = 1
        while len(vecs) > 1:
            m = (lane & t) != 0
            vecs = [jnp.where(m, vecs[i + 1] + xlane(vecs[i + 1], t),
                              vecs[i] + xlane(vecs[i], t))
                    for i in range(0, len(vecs), 2)]
            t *= 2
        out_v[pl.ds(g * L, L)] = vecs[0]
        return _

    lax.fori_loop(0, NG, group, None)

    pltpu.sync_copy(out_v, out_hbm.at[pl.ds(base, BPW)])


@jax.jit
def _scores(user_ids, item_ids, utab_t, itab_t, ulast, ilast):
    mesh = plsc.VectorSubcoreMesh(core_axis_name="c", subcore_axis_name="s")
    kern = functools.partial(
        pl.kernel,
        out_type=jax.ShapeDtypeStruct((BATCH,), jnp.float32),
        mesh=mesh,
        compiler_params=pltpu.CompilerParams(needs_layout_passes=False),
        scratch_types=[
            pltpu.VMEM((BPW,), jnp.int32),
            pltpu.VMEM((BPW,), jnp.int32),
            pltpu.VMEM((RING, DIM, 128), jnp.float32),
            pltpu.VMEM((RING, DIM, 128), jnp.float32),
            pltpu.VMEM(((NROW - NFULL) * DIM,), jnp.float32),
            pltpu.VMEM(((NROW - NFULL) * DIM,), jnp.float32),
            pltpu.VMEM((BPW,), jnp.float32),
            pltpu.SemaphoreType.DMA,
        ],
    )(_body)
    return kern(user_ids, item_ids, utab_t, itab_t, ulast, ilast)


def kernel(user_ids, item_ids, user_table, item_table):
    return _scores(user_ids.astype(jnp.int32), item_ids.astype(jnp.int32),
                   user_table.T, item_table.T,
                   user_table[NFULL:].reshape(-1),
                   item_table[NFULL:].reshape(-1))


# speedup vs baseline: 25.9741x; 1.0033x over previous
import functools

import jax
import jax.numpy as jnp
from jax import lax
from jax.experimental import pallas as pl
from jax.experimental.pallas import tpu as pltpu
from jax.experimental.pallas import tpu_sc as plsc

DIM = 64
BATCH = 16384
NROW = 1000000
NFULL = (NROW // 128) * 128
BMAX = NROW // 128 - 1
NC = 2
NS = 16
NW = NC * NS
BPW = BATCH // NW
L = 16
NG = BPW // L
RING = 6


def _body(uid_hbm, iid_hbm, utab_hbm, itab_hbm, ulast_hbm, ilast_hbm, out_hbm,
          uids_v, iids_v, ublk_v, iblk_v, ulast_v, ilast_v, out_v, sem):
    wid = lax.axis_index("s") * NC + lax.axis_index("c")
    base = wid * BPW

    pltpu.sync_copy(uid_hbm.at[pl.ds(base, BPW)], uids_v)
    pltpu.sync_copy(iid_hbm.at[pl.ds(base, BPW)], iids_v)
    pltpu.sync_copy(ulast_hbm, ulast_v)
    pltpu.sync_copy(ilast_hbm, ilast_v)

    lane = lax.iota(jnp.int32, L)
    dvec = [jnp.full((L,), q * L, jnp.int32) + lane for q in range(DIM // L)]

    def xlane(v, t):
        return v.at[lane ^ t].get(mode="promise_in_bounds")

    def splat(s):
        return jnp.full((L,), 0, jnp.int32) + s

    def fire(vec, k, tab, blk):
        b = jnp.minimum(vec[k] >> 7, BMAX)
        off = pl.multiple_of(b * 128, 128)
        pltpu.async_copy(tab.at[:, pl.ds(off, 128)], blk.at[k % RING], sem)

    def drain(tab, blk, k):
        pltpu.make_async_copy(tab.at[:, pl.ds(0, 128)],
                              blk.at[k % RING], sem).wait()

    def col(vec, k, blk, lastf):
        vid = vec[k]
        c = splat(vid & 127)
        rel = splat(jnp.maximum(vid - NFULL, 0) * DIM)
        m = splat(vid) >= NFULL
        out = []
        for q in range(DIM // L):
            hbm_q = plsc.load_gather(blk, [splat(k % RING), dvec[q], c])
            last_q = plsc.load_gather(lastf, [rel + dvec[q]])
            out.append(jnp.where(m, last_q, hbm_q))
        return out

    def group(g, _):
        uvec = uids_v[pl.ds(g * L, L)]
        ivec = iids_v[pl.ds(g * L, L)]
        for k in range(RING):
            fire(uvec, k, utab_hbm, ublk_v)
            fire(ivec, k, itab_hbm, iblk_v)
        vecs = []
        for k in range(L):
            drain(utab_hbm, ublk_v, k)
            drain(itab_hbm, iblk_v, k)
            u = col(uvec, k, ublk_v, ulast_v)
            v = col(ivec, k, iblk_v, ilast_v)
            acc = None
            for q in range(DIM // L):
                prod = u[q] * v[q]
                acc = prod if acc is None else acc + prod
            vecs.append(acc)
            if k + RING < L:
                fire(uvec, k + RING, utab_hbm, ublk_v)
                fire(ivec, k + RING, itab_hbm, iblk_v)
        t = 1
        while len(vecs) > 1:
            m = (lane & t) != 0
            vecs = [jnp.where(m, vecs[i + 1] + xlane(vecs[i + 1], t),
                              vecs[i] + xlane(vecs[i], t))
                    for i in range(0, len(vecs), 2)]
            t *= 2
        out_v[pl.ds(g * L, L)] = vecs[0]
        return _

    lax.fori_loop(0, NG, group, None)

    pltpu.sync_copy(out_v, out_hbm.at[pl.ds(base, BPW)])


@jax.jit
def _scores(user_ids, item_ids, utab_t, itab_t, ulast, ilast):
    mesh = plsc.VectorSubcoreMesh(core_axis_name="c", subcore_axis_name="s")
    kern = functools.partial(
        pl.kernel,
        out_type=jax.ShapeDtypeStruct((BATCH,), jnp.float32),
        mesh=mesh,
        compiler_params=pltpu.CompilerParams(needs_layout_passes=False),
        scratch_types=[
            pltpu.VMEM((BPW,), jnp.int32),
            pltpu.VMEM((BPW,), jnp.int32),
            pltpu.VMEM((RING, DIM, 128), jnp.float32),
            pltpu.VMEM((RING, DIM, 128), jnp.float32),
            pltpu.VMEM(((NROW - NFULL) * DIM,), jnp.float32),
            pltpu.VMEM(((NROW - NFULL) * DIM,), jnp.float32),
            pltpu.VMEM((BPW,), jnp.float32),
            pltpu.SemaphoreType.DMA,
        ],
    )(_body)
    return kern(user_ids, item_ids, utab_t, itab_t, ulast, ilast)


def kernel(user_ids, item_ids, user_table, item_table):
    return _scores(user_ids.astype(jnp.int32), item_ids.astype(jnp.int32),
                   user_table.T, item_table.T,
                   user_table[NFULL:].reshape(-1),
                   item_table[NFULL:].reshape(-1))


# speedup vs baseline: 26.5898x; 1.0237x over previous
import functools

import jax
import jax.numpy as jnp
from jax import lax
from jax.experimental import pallas as pl
from jax.experimental.pallas import tpu as pltpu
from jax.experimental.pallas import tpu_sc as plsc

DIM = 64
BATCH = 16384
NROW = 1000000
NFULL = (NROW // 128) * 128
BMAX = NROW // 128 - 1
NC = 2
NS = 16
NW = NC * NS
BPW = BATCH // NW
L = 16
NG = BPW // L
RING = 6


def _body(uid_hbm, iid_hbm, utab_hbm, itab_hbm, ulast_hbm, ilast_hbm, out_hbm,
          uids_v, iids_v, ublk_v, iblk_v, ulast_v, ilast_v, out_v, sem):
    wid = lax.axis_index("s") * NC + lax.axis_index("c")
    base = wid * BPW

    pltpu.sync_copy(uid_hbm.at[pl.ds(base, BPW)], uids_v)
    pltpu.sync_copy(iid_hbm.at[pl.ds(base, BPW)], iids_v)
    pltpu.sync_copy(ulast_hbm, ulast_v)
    pltpu.sync_copy(ilast_hbm, ilast_v)

    lane = lax.iota(jnp.int32, L)
    dvec = [jnp.full((L,), q * L, jnp.int32) + lane for q in range(DIM // L)]

    def xlane(v, t):
        return v.at[lane ^ t].get(mode="promise_in_bounds")

    def splat(s):
        return jnp.full((L,), 0, jnp.int32) + s

    def fire(vec, k, tab, blk):
        b = jnp.minimum(vec[k] >> 7, BMAX)
        off = pl.multiple_of(b * 128, 128)
        pltpu.async_copy(tab.at[:, pl.ds(off, 128)], blk.at[k % RING], sem)

    def drain(tab, blk, k):
        pltpu.make_async_copy(tab.at[:, pl.ds(0, 128)],
                              blk.at[k % RING], sem).wait()

    def col(vec, k, blk, lastf):
        vid = vec[k]
        c = splat(vid & 127)
        rel = splat(jnp.maximum(vid - NFULL, 0) * DIM)
        m = splat(vid) >= NFULL
        out = []
        for q in range(DIM // L):
            hbm_q = plsc.load_gather(blk, [splat(k % RING), dvec[q], c])
            last_q = plsc.load_gather(lastf, [rel + dvec[q]])
            out.append(jnp.where(m, last_q, hbm_q))
        return out

    def prologue():
        uvec = uids_v[pl.ds(0, L)]
        ivec = iids_v[pl.ds(0, L)]
        for k in range(RING):
            fire(uvec, k, utab_hbm, ublk_v)
            fire(ivec, k, itab_hbm, iblk_v)

    def group(g, _):
        uvec = uids_v[pl.ds(g * L, L)]
        ivec = iids_v[pl.ds(g * L, L)]
        nxt = jnp.minimum(g + 1, NG - 1) * L
        uvec_n = uids_v[pl.ds(nxt, L)]
        ivec_n = iids_v[pl.ds(nxt, L)]
        vecs = []
        for k in range(L):
            drain(utab_hbm, ublk_v, k)
            drain(itab_hbm, iblk_v, k)
            u = col(uvec, k, ublk_v, ulast_v)
            v = col(ivec, k, iblk_v, ilast_v)
            acc = None
            for q in range(DIM // L):
                prod = u[q] * v[q]
                acc = prod if acc is None else acc + prod
            vecs.append(acc)
            if k + RING < L:
                fire(uvec, k + RING, utab_hbm, ublk_v)
                fire(ivec, k + RING, itab_hbm, iblk_v)
            else:
                @pl.when(g < NG - 1)
                def _fire_next():
                    fire(uvec_n, k + RING - L, utab_hbm, ublk_v)
                    fire(ivec_n, k + RING - L, itab_hbm, iblk_v)
        t = 1
        while len(vecs) > 1:
            m = (lane & t) != 0
            vecs = [jnp.where(m, vecs[i + 1] + xlane(vecs[i + 1], t),
                              vecs[i] + xlane(vecs[i], t))
                    for i in range(0, len(vecs), 2)]
            t *= 2
        out_v[pl.ds(g * L, L)] = vecs[0]
        return _

    prologue()
    lax.fori_loop(0, NG, group, None)

    pltpu.sync_copy(out_v, out_hbm.at[pl.ds(base, BPW)])


@jax.jit
def _scores(user_ids, item_ids, utab_t, itab_t, ulast, ilast):
    mesh = plsc.VectorSubcoreMesh(core_axis_name="c", subcore_axis_name="s")
    kern = functools.partial(
        pl.kernel,
        out_type=jax.ShapeDtypeStruct((BATCH,), jnp.float32),
        mesh=mesh,
        compiler_params=pltpu.CompilerParams(needs_layout_passes=False),
        scratch_types=[
            pltpu.VMEM((BPW,), jnp.int32),
            pltpu.VMEM((BPW,), jnp.int32),
            pltpu.VMEM((RING, DIM, 128), jnp.float32),
            pltpu.VMEM((RING, DIM, 128), jnp.float32),
            pltpu.VMEM(((NROW - NFULL) * DIM,), jnp.float32),
            pltpu.VMEM(((NROW - NFULL) * DIM,), jnp.float32),
            pltpu.VMEM((BPW,), jnp.float32),
            pltpu.SemaphoreType.DMA,
        ],
    )(_body)
    return kern(user_ids, item_ids, utab_t, itab_t, ulast, ilast)


def kernel(user_ids, item_ids, user_table, item_table):
    return _scores(user_ids.astype(jnp.int32), item_ids.astype(jnp.int32),
                   user_table.T, item_table.T,
                   user_table[NFULL:].reshape(-1),
                   item_table[NFULL:].reshape(-1))


# speedup vs baseline: 27.2906x; 1.0264x over previous
import functools

import jax
import jax.numpy as jnp
from jax import lax
from jax.experimental import pallas as pl
from jax.experimental.pallas import tpu as pltpu
from jax.experimental.pallas import tpu_sc as plsc

DIM = 64
BATCH = 16384
NROW = 1000000
NFULL = (NROW // 128) * 128
BMAX = NROW // 128 - 1
NC = 2
NS = 16
NW = NC * NS
BPW = BATCH // NW
L = 16
NG = BPW // L
RING = 4


def _body(uid_hbm, iid_hbm, utab_hbm, itab_hbm, ulast_hbm, ilast_hbm, out_hbm,
          uids_v, iids_v, ublk_v, iblk_v, ulast_v, ilast_v, out_v, sem):
    wid = lax.axis_index("s") * NC + lax.axis_index("c")
    base = wid * BPW

    pltpu.sync_copy(uid_hbm.at[pl.ds(base, BPW)], uids_v)
    pltpu.sync_copy(iid_hbm.at[pl.ds(base, BPW)], iids_v)
    pltpu.sync_copy(ulast_hbm, ulast_v)
    pltpu.sync_copy(ilast_hbm, ilast_v)

    lane = lax.iota(jnp.int32, L)
    dvec = [jnp.full((L,), q * L, jnp.int32) + lane for q in range(DIM // L)]

    def xlane(v, t):
        return v.at[lane ^ t].get(mode="promise_in_bounds")

    def splat(s):
        return jnp.full((L,), 0, jnp.int32) + s

    def fire(vec, k, tab, blk):
        b = jnp.minimum(vec[k] >> 7, BMAX)
        off = pl.multiple_of(b * 128, 128)
        pltpu.async_copy(tab.at[:, pl.ds(off, 128)], blk.at[k % RING], sem)

    def drain(tab, blk, k):
        pltpu.make_async_copy(tab.at[:, pl.ds(0, 128)],
                              blk.at[k % RING], sem).wait()

    def col(vec, k, blk, lastf):
        vid = vec[k]
        c = splat(vid & 127)
        rel = splat(jnp.maximum(vid - NFULL, 0) * DIM)
        m = splat(vid) >= NFULL
        out = []
        for q in range(DIM // L):
            hbm_q = plsc.load_gather(blk, [splat(k % RING), dvec[q], c])
            last_q = plsc.load_gather(lastf, [rel + dvec[q]])
            out.append(jnp.where(m, last_q, hbm_q))
        return out

    def prologue():
        uvec = uids_v[pl.ds(0, L)]
        ivec = iids_v[pl.ds(0, L)]
        for k in range(RING):
            fire(uvec, k, utab_hbm, ublk_v)
            fire(ivec, k, itab_hbm, iblk_v)

    def group(g, _):
        uvec = uids_v[pl.ds(g * L, L)]
        ivec = iids_v[pl.ds(g * L, L)]
        nxt = jnp.minimum(g + 1, NG - 1) * L
        uvec_n = uids_v[pl.ds(nxt, L)]
        ivec_n = iids_v[pl.ds(nxt, L)]
        vecs = []
        for k in range(L):
            drain(utab_hbm, ublk_v, k)
            drain(itab_hbm, iblk_v, k)
            u = col(uvec, k, ublk_v, ulast_v)
            v = col(ivec, k, iblk_v, ilast_v)
            acc = None
            for q in range(DIM // L):
                prod = u[q] * v[q]
                acc = prod if acc is None else acc + prod
            vecs.append(acc)
            if k + RING < L:
                fire(uvec, k + RING, utab_hbm, ublk_v)
                fire(ivec, k + RING, itab_hbm, iblk_v)
            else:
                @pl.when(g < NG - 1)
                def _fire_next():
                    fire(uvec_n, k + RING - L, utab_hbm, ublk_v)
                    fire(ivec_n, k + RING - L, itab_hbm, iblk_v)
        t = 1
        while len(vecs) > 1:
            m = (lane & t) != 0
            vecs = [jnp.where(m, vecs[i + 1] + xlane(vecs[i + 1], t),
                              vecs[i] + xlane(vecs[i], t))
                    for i in range(0, len(vecs), 2)]
            t *= 2
        out_v[pl.ds(g * L, L)] = vecs[0]
        return _

    prologue()
    lax.fori_loop(0, NG, group, None)

    pltpu.sync_copy(out_v, out_hbm.at[pl.ds(base, BPW)])


@jax.jit
def _scores(user_ids, item_ids, utab_t, itab_t, ulast, ilast):
    mesh = plsc.VectorSubcoreMesh(core_axis_name="c", subcore_axis_name="s")
    kern = functools.partial(
        pl.kernel,
        out_type=jax.ShapeDtypeStruct((BATCH,), jnp.float32),
        mesh=mesh,
        compiler_params=pltpu.CompilerParams(needs_layout_passes=False),
        scratch_types=[
            pltpu.VMEM((BPW,), jnp.int32),
            pltpu.VMEM((BPW,), jnp.int32),
            pltpu.VMEM((RING, DIM, 128), jnp.float32),
            pltpu.VMEM((RING, DIM, 128), jnp.float32),
            pltpu.VMEM(((NROW - NFULL) * DIM,), jnp.float32),
            pltpu.VMEM(((NROW - NFULL) * DIM,), jnp.float32),
            pltpu.VMEM((BPW,), jnp.float32),
            pltpu.SemaphoreType.DMA,
        ],
    )(_body)
    return kern(user_ids, item_ids, utab_t, itab_t, ulast, ilast)


def kernel(user_ids, item_ids, user_table, item_table):
    return _scores(user_ids.astype(jnp.int32), item_ids.astype(jnp.int32),
                   user_table.T, item_table.T,
                   user_table[NFULL:].reshape(-1),
                   item_table[NFULL:].reshape(-1))
